# Initial kernel scaffold; baseline (speedup 1.0000x reference)
import jax
import jax.numpy as jnp
from jax.experimental import pallas as pl


def kernel(user_word, src0, dst0, src1, dst1, labels, word_table, w1, bias1, attn1, w2, bias2, attn2, gamma, beta, Wg1, bg1, Wg2, bg2):
    raise NotImplementedError("write your pallas kernel here")


# speedup vs baseline: 1.3077x; 1.3077x over previous
import functools

import jax
import jax.numpy as jnp
from jax import lax
from jax.experimental import pallas as pl
from jax.experimental.pallas import tpu as pltpu
from jax.experimental.pallas import tpu_sc as plsc

N0, N1, N2 = 10000, 5000, 2500
E0, E1 = 160000, 80000
V, WD = 50000, 64
USER, HID, OUT = 256, 256, 128

NC, NS = 2, 16
NW = NC * NS
CH = 128

OFF_SRC0 = 0
OFF_DST0 = 10240
OFF_SRC1 = 15360
OFF_DST1 = 20480
DEG_PAD = 23040
N1P = 5120
N2P = 2560


def _mesh():
    return plsc.VectorSubcoreMesh(core_axis_name="c", subcore_axis_name="s")


def _wid():
    cid = lax.axis_index("c")
    sid = lax.axis_index("s")
    return cid, sid, sid * NC + cid


def _deg_call(src0, dst0, src1, dst1):
    per = DEG_PAD // NS

    @functools.partial(
        pl.kernel,
        out_type=jax.ShapeDtypeStruct((NC * DEG_PAD,), jnp.float32),
        mesh=_mesh(),
        scratch_types=[
            pltpu.VMEM((CH,), jnp.int32),
            pltpu.VMEM((CH,), jnp.float32),
            pltpu.VMEM((per,), jnp.float32),
            pltpu.VMEM_SHARED((DEG_PAD,), jnp.float32),
        ],
        name="degrees",
    )
    def k(s0, d0, s1, d1, out, idx_v, ones_v, zbuf, hist):
        cid, sid, wid = _wid()
        for t in range(CH // 16):
            ones_v[pl.ds(t * 16, 16)] = jnp.ones((16,), jnp.float32)

        def zb(i, _):
            zbuf[pl.ds(i * 16, 16)] = jnp.zeros((16,), jnp.float32)
            return 0

        lax.fori_loop(0, per // 16, zb, 0)
        pltpu.sync_copy(zbuf, hist.at[pl.ds(sid * per, per)])
        plsc.subcore_barrier()
        for arr, off, ne in ((s0, OFF_SRC0, E0), (d0, OFF_DST0, E0),
                             (s1, OFF_SRC1, E1), (d1, OFF_DST1, E1)):
            nch = ne // CH

            def body(kk, _, arr=arr, off=off):
                c = wid + NW * kk
                pltpu.sync_copy(arr.at[pl.ds(pl.multiple_of(c * CH, 8), CH)], idx_v)
                for t in range(CH // 16):
                    idx_v[pl.ds(t * 16, 16)] = idx_v[pl.ds(t * 16, 16)] + off
                pltpu.sync_copy(ones_v, hist.at[idx_v], add=True)
                return 0

            lax.fori_loop(0, (nch + NW - 1 - wid) // NW, body, 0)
        plsc.subcore_barrier()
        pltpu.sync_copy(hist.at[pl.ds(sid * per, per)], zbuf)
        pltpu.sync_copy(zbuf, out.at[pl.ds(pl.multiple_of(cid * DEG_PAD + sid * per, 8), per)])

    return k(src0, dst0, src1, dst1)


UPC = 8


def _emb_call(word_table, uw):
    nch = N0 // UPC

    @functools.partial(
        pl.kernel,
        out_type=jax.ShapeDtypeStruct((N0, WD), jnp.float32),
        mesh=_mesh(),
        scratch_types=[
            pltpu.VMEM((UPC * 64,), jnp.int32),
            pltpu.VMEM((UPC * 64, WD), jnp.float32),
            pltpu.VMEM((UPC, WD), jnp.float32),
            pltpu.SemaphoreType.DMA,
        ],
        compiler_params=pltpu.CompilerParams(use_tc_tiling_on_sc=False),
        name="embed_sum",
    )
    def k(wt, idx_hbm, out, idx_v, buf, e_buf, sem):
        cid, sid, wid = _wid()

        def body(kk, _):
            c = wid + NW * kk
            pltpu.sync_copy(idx_hbm.at[pl.ds(pl.multiple_of(c * UPC * 64, 8), UPC * 64)], idx_v)
            pltpu.async_copy(wt.at[idx_v], buf, sem).wait()

            def ubody(u, _):
                def wbody(w, accs):
                    r = u * 64 + w
                    return tuple(accs[j] + buf[r, pl.ds(j * 16, 16)] for j in range(WD // 16))

                accs = lax.fori_loop(0, 64, wbody,
                                     tuple(jnp.zeros((16,), jnp.float32) for _ in range(WD // 16)))
                for j in range(WD // 16):
                    e_buf[u, pl.ds(j * 16, 16)] = accs[j]
                return 0

            lax.fori_loop(0, UPC, ubody, 0)
            pltpu.sync_copy(e_buf, out.at[pl.ds(c * UPC, UPC)])
            return 0

        lax.fori_loop(0, (nch + NW - 1 - wid) // NW, body, 0)

    return k(word_table, uw)


CHS = 4000
BLK = 128


def _scatter_call(h, src, dst, E, npad, D, name):
    nch = E // CHS
    rpt = npad // NW
    arows = rpt + 8
    jb = CHS + BLK
    cap = CHS + 2 * BLK

    @functools.partial(
        pl.kernel,
        out_type=jax.ShapeDtypeStruct((npad * D,), jnp.float32),
        mesh=_mesh(),
        scratch_types=[
            pltpu.VMEM((CHS,), jnp.int32),
            pltpu.VMEM((CHS,), jnp.int32),
            pltpu.VMEM((cap,), jnp.int32),
            pltpu.VMEM((cap,), jnp.int32),
            pltpu.VMEM((BLK, D), jnp.float32),
            pltpu.VMEM((arows * D,), jnp.float32),
            pltpu.SemaphoreType.DMA,
        ],
        compiler_params=pltpu.CompilerParams(needs_layout_passes=False),
        name=name,
    )
    def k(h_hbm, src_hbm, dst_hbm, out, sidx, didx, cl_src, cl_dst, rows, acc, sem):
        cid, sid, wid = _wid()
        lo = wid * rpt
        lanes = jnp.arange(16, dtype=jnp.int32)

        def za(i, _):
            acc[pl.ds(i * 16, 16)] = jnp.zeros((16,), jnp.float32)
            return 0

        lax.fori_loop(0, (arows * D) // 16, za, 0)

        def zc(i, _):
            cl_src[pl.ds(i * 16, 16)] = jnp.zeros((16,), jnp.int32)
            cl_dst[pl.ds(i * 16, 16)] = jnp.zeros((16,), jnp.int32)
            return 0

        lax.fori_loop(0, cap // 16, zc, 0)

        def chunk(c, _):
            off = pl.multiple_of(c * CHS, 8)
            pltpu.sync_copy(src_hbm.at[pl.ds(off, CHS)], sidx)
            pltpu.sync_copy(dst_hbm.at[pl.ds(off, CHS)], didx)

            def grp(g, p):
                d = didx[pl.ds(g * 16, 16)]
                s = sidx[pl.ds(g * 16, 16)]
                dl = d - lo
                m = (dl >= 0) & (dl < rpt)
                mi = m.astype(jnp.int32)
                cs = plsc.cumsum(mi)
                pos = jnp.where(m, p + cs - 1, jb + lanes)
                plsc.store_scatter(cl_dst, [pos], dl)
                plsc.store_scatter(cl_src, [pos], s)
                return p + cs[15]

            ptr = lax.fori_loop(0, CHS // 16, grp, 0)
            nbt = (ptr + BLK - 1) // BLK

            def blk(b, _):
                pltpu.async_copy(h_hbm.at[cl_src.at[pl.ds(b * BLK, BLK)]],
                                 rows, sem).wait()
                limit = ptr - b * BLK

                def radd(g, _):
                    dlv = cl_dst[pl.ds(b * BLK + g * 16, 16)]
                    for i in range(16):
                        @pl.when(g * 16 + i < limit)
                        def _(i=i):
                            dl = dlv[i]
                            for j in range(D // 16):
                                plsc.addupdate(
                                    acc.at[pl.ds(dl * D + j * 16, 16)],
                                    rows[g * 16 + i, pl.ds(j * 16, 16)])
                    return 0

                lax.fori_loop(0, BLK // 16, radd, 0)
                return 0

            lax.fori_loop(0, nbt, blk, 0)
            return 0

        lax.fori_loop(0, nch, chunk, 0)
        pltpu.sync_copy(acc.at[pl.ds(0, rpt * D)],
                        out.at[pl.ds(pl.multiple_of(lo * D, 8), rpt * D)])

    return k(h, src, dst)


def _x_stats_call(e, w1, w2, b1, b2):
    BR = 1000

    def body(e_ref, w1_ref, w2_ref, b1_ref, b2_ref, x_ref, st_ref):
        t = jnp.dot(e_ref[...], w1_ref[...], preferred_element_type=jnp.float32)
        c = (64.0 * jnp.dot(b1_ref[...], w2_ref[...], preferred_element_type=jnp.float32)
             + 8.0 * b2_ref[...])
        xb = jnp.dot(t, w2_ref[...], preferred_element_type=jnp.float32) + c
        x_ref[...] = xb
        s = jnp.sum(xb, axis=0, keepdims=True)
        q = jnp.sum(xb * xb, axis=0, keepdims=True)
        upd = jnp.concatenate([s, q, jnp.zeros((6, USER), jnp.float32)], axis=0)

        @pl.when(pl.program_id(0) == 0)
        def _():
            st_ref[...] = upd

        @pl.when(pl.program_id(0) != 0)
        def _():
            st_ref[...] = st_ref[...] + upd

    return pl.pallas_call(
        body,
        grid=(N0 // BR,),
        in_specs=[
            pl.BlockSpec((BR, WD), lambda i: (i, 0)),
            pl.BlockSpec((WD, WD), lambda i: (0, 0)),
            pl.BlockSpec((WD, USER), lambda i: (0, 0)),
            pl.BlockSpec((1, WD), lambda i: (0, 0)),
            pl.BlockSpec((1, USER), lambda i: (0, 0)),
        ],
        out_specs=[
            pl.BlockSpec((BR, USER), lambda i: (i, 0)),
            pl.BlockSpec((8, USER), lambda i: (0, 0)),
        ],
        out_shape=[
            jax.ShapeDtypeStruct((N0, USER), jnp.float32),
            jax.ShapeDtypeStruct((8, USER), jnp.float32),
        ],
    )(e, w1, w2, b1, b2)


def _h1_call(x, stats, deg0c, gamma, beta, Wg1):
    BR = 1000

    def body(x_ref, st_ref, dg_ref, g_ref, b_ref, w_ref, o_ref):
        mean = st_ref[0:1, :] * (1.0 / N0)
        msq = st_ref[1:2, :] * (1.0 / N0)
        var = msq - mean * mean
        s = g_ref[...] * lax.rsqrt(var + 1e-5)
        t = b_ref[...] - mean * s
        xn = x_ref[...] * s + t
        h = jnp.dot(xn, w_ref[...], preferred_element_type=jnp.float32)
        d = dg_ref[0] + dg_ref[1]
        o_ref[...] = h * lax.rsqrt(jnp.maximum(d, 1.0))

    return pl.pallas_call(
        body,
        grid=(N0 // BR,),
        in_specs=[
            pl.BlockSpec((BR, USER), lambda i: (i, 0)),
            pl.BlockSpec((8, USER), lambda i: (0, 0)),
            pl.BlockSpec((2, BR, 1), lambda i: (0, i, 0)),
            pl.BlockSpec((1, USER), lambda i: (0, 0)),
            pl.BlockSpec((1, USER), lambda i: (0, 0)),
            pl.BlockSpec((USER, HID), lambda i: (0, 0)),
        ],
        out_specs=pl.BlockSpec((BR, HID), lambda i: (i, 0)),
        out_shape=jax.ShapeDtypeStruct((N0, HID), jnp.float32),
    )(x, stats, deg0c, gamma, beta, Wg1)


def _x2h2_call(agg1, degin, degout, bg1, Wg2):
    def body(p_ref, di_ref, do_ref, b_ref, w_ref, o_ref):
        agg = p_ref[...]
        din = di_ref[0] + di_ref[1]
        x1 = jnp.maximum(agg * lax.rsqrt(jnp.maximum(din, 1.0)) + b_ref[...], 0.0)
        h = jnp.dot(x1, w_ref[...], preferred_element_type=jnp.float32)
        dout = do_ref[0] + do_ref[1]
        o_ref[...] = h * lax.rsqrt(jnp.maximum(dout, 1.0))

    return pl.pallas_call(
        body,
        out_shape=jax.ShapeDtypeStruct((N1P, OUT), jnp.float32),
    )(agg1, degin, degout, bg1, Wg2)


def _out_call(agg2, degin, bg2):
    def body(p_ref, di_ref, b_ref, o_ref):
        agg = p_ref[...]
        d = di_ref[0] + di_ref[1]
        o_ref[...] = jnp.maximum(agg * lax.rsqrt(jnp.maximum(d, 1.0)) + b_ref[...], 0.0)

    return pl.pallas_call(
        body,
        out_shape=jax.ShapeDtypeStruct((N2P, OUT), jnp.float32),
    )(agg2, degin, bg2)


def kernel(user_word, src0, dst0, src1, dst1, labels, word_table, w1, bias1,
           attn1, w2, bias2, attn2, gamma, beta, Wg1, bg1, Wg2, bg2):
    uw = user_word.reshape(N0 * 64).astype(jnp.int32)
    src0 = src0.astype(jnp.int32)
    dst0 = dst0.astype(jnp.int32)
    src1 = src1.astype(jnp.int32)
    dst1 = dst1.astype(jnp.int32)

    deg = _deg_call(src0, dst0, src1, dst1).reshape(NC, DEG_PAD)
    e = _emb_call(word_table, uw)
    x, stats = _x_stats_call(e, w1, w2, bias1.reshape(1, WD), bias2.reshape(1, USER))
    deg0c = deg[:, :N0].reshape(NC, N0, 1)
    h1 = _h1_call(x, stats, deg0c, gamma.reshape(1, USER), beta.reshape(1, USER), Wg1)
    agg1 = _scatter_call(h1, src0, dst0, E0, N1P, HID, "seg_sum1").reshape(N1P, HID)
    degin0 = deg[:, OFF_DST0:OFF_DST0 + N1P].reshape(NC, N1P, 1)
    degout1 = deg[:, OFF_SRC1:OFF_SRC1 + N1P].reshape(NC, N1P, 1)
    h2 = _x2h2_call(agg1, degin0, degout1, bg1.reshape(1, HID), Wg2)
    agg2 = _scatter_call(h2, src1, dst1, E1, N2P, OUT, "seg_sum2").reshape(N2P, OUT)
    degin1 = deg[:, OFF_DST1:OFF_DST1 + N2P].reshape(NC, N2P, 1)
    xo = _out_call(agg2, degin1, bg2.reshape(1, OUT))
    return (xo[:N2], labels)
